# baseline (speedup 1.0000x reference)
import functools

import jax
import jax.numpy as jnp
from jax import lax
from jax.experimental import pallas as pl
from jax.experimental.pallas import tpu as pltpu

SEQ = 100000
LANE = 128
R0_MAX = SEQ // LANE - 1


def _tc_lookup(t_ref, pose_ref, out_ref, col_v, sem):
    t = t_ref[0]
    i0 = (t * jnp.float32(SEQ - 1) + jnp.float32(0.5)).astype(jnp.int32)
    r0 = jnp.minimum(jnp.maximum((i0 - 8) >> 7, 0), R0_MAX)
    gidx = (r0 * LANE
            + lax.broadcasted_iota(jnp.int32, (2, LANE), 0) * LANE
            + lax.broadcasted_iota(jnp.int32, (2, LANE), 1))
    step = jnp.float32(1.0) / jnp.float32(SEQ - 1)
    lin = jnp.where(gidx == SEQ - 1, jnp.float32(1.0),
                    gidx.astype(jnp.float32) * step)
    d = jnp.abs(lin - t)
    m = jnp.min(d)
    i = jnp.min(jnp.where(d == m, gidx.astype(jnp.float32),
                          jnp.float32(2**30))).astype(jnp.int32)

    c0 = pl.multiple_of((i >> 7) << 7, LANE)
    lane = i - c0
    cp = pltpu.make_async_copy(pose_ref.at[:, :, pl.ds(c0, LANE)], col_v, sem)
    cp.start()
    cp.wait()
    onehot = lax.broadcasted_iota(jnp.int32, (3, 75, LANE), 2) == lane
    out_ref[...] = jnp.sum(jnp.where(onehot, col_v[...], 0.0), axis=2)


_lookup = functools.partial(
    pl.pallas_call,
    out_shape=jax.ShapeDtypeStruct((3, 75), jnp.float32),
    in_specs=[
        pl.BlockSpec(memory_space=pltpu.SMEM),
        pl.BlockSpec(memory_space=pltpu.MemorySpace.HBM),
    ],
    scratch_shapes=[
        pltpu.VMEM((3, 75, LANE), jnp.float32),
        pltpu.SemaphoreType.DMA,
    ],
)(_tc_lookup)


def kernel(pose_params, time_point):
    pose_t = jnp.transpose(pose_params, (2, 1, 0))
    t2 = jnp.reshape(time_point, (1,))
    out = _lookup(t2, pose_t)
    return out.T


# speedup vs baseline: 1.0024x; 1.0024x over previous
import functools

import jax
import jax.numpy as jnp
from jax import lax
from jax.experimental import pallas as pl
from jax.experimental.pallas import tpu as pltpu

SEQ = 100000
LANE = 128
R0_MAX = SEQ // LANE - 1


def _tc_lookup(t_ref, pose_ref, out_ref, col_v, sem):
    t = t_ref[...]
    i0 = (t * jnp.float32(SEQ - 1) + jnp.float32(0.5)).astype(jnp.int32)
    r0 = jnp.minimum(jnp.maximum((i0 - 8) >> 7, 0), R0_MAX)
    gidx = (r0 * LANE
            + lax.broadcasted_iota(jnp.int32, (2, LANE), 0) * LANE
            + lax.broadcasted_iota(jnp.int32, (2, LANE), 1))
    step = jnp.float32(1.0) / jnp.float32(SEQ - 1)
    lin = jnp.where(gidx == SEQ - 1, jnp.float32(1.0),
                    gidx.astype(jnp.float32) * step)
    d = jnp.abs(lin - t)
    m = jnp.min(d)
    i = jnp.min(jnp.where(d == m, gidx.astype(jnp.float32),
                          jnp.float32(2**30))).astype(jnp.int32)

    c0 = pl.multiple_of((i >> 7) << 7, LANE)
    lane = i - c0
    cp = pltpu.make_async_copy(pose_ref.at[:, :, pl.ds(c0, LANE)], col_v, sem)
    cp.start()
    cp.wait()
    onehot = lax.broadcasted_iota(jnp.int32, (3, 75, LANE), 2) == lane
    out_ref[...] = jnp.sum(jnp.where(onehot, col_v[...], 0.0), axis=2)


_lookup = functools.partial(
    pl.pallas_call,
    out_shape=jax.ShapeDtypeStruct((3, 75), jnp.float32),
    in_specs=[
        pl.BlockSpec(memory_space=pltpu.SMEM),
        pl.BlockSpec(memory_space=pltpu.MemorySpace.HBM),
    ],
    scratch_shapes=[
        pltpu.VMEM((3, 75, LANE), jnp.float32),
        pltpu.SemaphoreType.DMA,
    ],
)(_tc_lookup)


def kernel(pose_params, time_point):
    pose_t = jnp.transpose(pose_params, (2, 1, 0))
    out = _lookup(time_point, pose_t)
    return out.T
